# baseline (speedup 1.0000x reference)
import jax
import jax.numpy as jnp
from jax import lax
from jax.experimental import pallas as pl
from jax.experimental.pallas import tpu as pltpu

VOCAB_PAD = 1024
DIM = 128
BM = 512


def _tc_lookup(hi_lo, idx3, m):
    nb = m // BM

    def body(idx_ref, tbl_ref, out_ref):
        idxv = idx_ref[0]
        iota = lax.broadcasted_iota(jnp.int32, (VOCAB_PAD, BM), 0)
        mask = iota == jnp.broadcast_to(idxv, (VOCAB_PAD, BM))
        oh = jnp.where(mask, jnp.float32(1), jnp.float32(0)).astype(jnp.bfloat16)
        dn = (((0,), (0,)), ((), ()))
        hi_part = lax.dot_general(
            oh, tbl_ref[pl.ds(0, VOCAB_PAD), :], dn,
            preferred_element_type=jnp.float32)
        lo_part = lax.dot_general(
            oh, tbl_ref[pl.ds(VOCAB_PAD, VOCAB_PAD), :], dn,
            preferred_element_type=jnp.float32)
        out_ref[...] = hi_part + lo_part

    return pl.pallas_call(
        body,
        grid=(nb,),
        in_specs=[
            pl.BlockSpec((1, 1, BM), lambda i: (i, 0, 0)),
            pl.BlockSpec((2 * VOCAB_PAD, DIM), lambda i: (0, 0)),
        ],
        out_specs=pl.BlockSpec((BM, DIM), lambda i: (i, 0)),
        out_shape=jax.ShapeDtypeStruct((m, DIM), jnp.float32),
    )(idx3, hi_lo)


def kernel(table, indices_tensor):
    batch, seq = indices_tensor.shape
    n = batch * seq
    tbl = jnp.zeros((VOCAB_PAD, DIM), jnp.float32).at[: table.shape[0]].set(table)
    hi_f32 = lax.reduce_precision(tbl, 8, 7)
    hi = hi_f32.astype(jnp.bfloat16)
    lo = (tbl - hi_f32).astype(jnp.bfloat16)
    hi_lo = jnp.concatenate([hi, lo], axis=0)
    idx3 = indices_tensor.reshape(n // BM, 1, BM).astype(jnp.int32)
    out = _tc_lookup(hi_lo, idx3, n)
    return out.reshape(batch, seq, DIM)


# speedup vs baseline: 4.0562x; 4.0562x over previous
import functools

import jax
import jax.numpy as jnp
from jax import lax
from jax.experimental import pallas as pl
from jax.experimental.pallas import tpu as pltpu
from jax.experimental.pallas import tpu_sc as plsc

VOCAB = 1002
DIM = 128
WIN = 128
NWORKERS = 32
NBUF = 4


def kernel(table, indices_tensor):
    batch, seq = indices_tensor.shape
    n = batch * seq
    nwin = n // WIN
    wpw = nwin // NWORKERS
    idx2d = indices_tensor.reshape(nwin, WIN).astype(jnp.int32)

    mesh = plsc.VectorSubcoreMesh(core_axis_name="c", subcore_axis_name="s")

    @functools.partial(
        pl.kernel,
        out_type=jax.ShapeDtypeStruct((n, DIM), table.dtype),
        mesh=mesh,
        scratch_types=[
            pltpu.VMEM_SHARED((VOCAB, DIM), jnp.float32),
            pltpu.VMEM((wpw, WIN), jnp.int32),
            pltpu.VMEM((NBUF, WIN, DIM), jnp.float32),
            pltpu.SemaphoreType.DMA,
            pltpu.SemaphoreType.DMA((NBUF,)),
            pltpu.SemaphoreType.DMA((NBUF,)),
        ],
    )
    def gather_kernel(table_hbm, idx_hbm, out_hbm, table_sh, idx_v, bufs,
                      isem, gsem, wsem):
        cid = lax.axis_index("c")
        sid = lax.axis_index("s")
        wid = sid * 2 + cid

        idx_cp = pltpu.async_copy(idx_hbm.at[pl.ds(wid * wpw, wpw)], idx_v, isem)

        @pl.when(sid == 0)
        def _():
            pltpu.sync_copy(table_hbm, table_sh)

        idx_cp.wait()
        plsc.subcore_barrier()

        base = wid * (wpw * WIN)

        @pl.loop(0, wpw, step=NBUF)
        def _(j0):
            gathers = []
            for b in range(NBUF):
                slot = bufs.at[b]

                @pl.when(j0 > 0)
                def _():
                    pltpu.make_async_copy(
                        slot, out_hbm.at[pl.ds(base, WIN)], wsem.at[b]).wait()

                src_tbl = table_sh if b % 2 == 0 else table_hbm
                gathers.append(pltpu.async_copy(
                    src_tbl.at[idx_v.at[j0 + b]], slot, gsem.at[b]))

            for b in range(NBUF):
                gathers[b].wait()
                pltpu.async_copy(
                    bufs.at[b], out_hbm.at[pl.ds(base + (j0 + b) * WIN, WIN)],
                    wsem.at[b])

        for b in range(NBUF):
            pltpu.make_async_copy(
                bufs.at[b], out_hbm.at[pl.ds(base, WIN)], wsem.at[b]).wait()

    out = gather_kernel(table, idx2d)
    return out.reshape(batch, seq, DIM)


# speedup vs baseline: 6.5655x; 1.6186x over previous
import functools

import jax
import jax.numpy as jnp
from jax import lax
from jax.experimental import pallas as pl
from jax.experimental.pallas import tpu as pltpu
from jax.experimental.pallas import tpu_sc as plsc

VOCAB = 1002
DIM = 128
WIN = 128
NWORKERS = 32
NBUF = 4


def kernel(table, indices_tensor):
    batch, seq = indices_tensor.shape
    n = batch * seq
    nwin = n // WIN
    wpw = nwin // NWORKERS
    idx2d = indices_tensor.reshape(nwin, WIN).astype(jnp.int32)

    mesh = plsc.VectorSubcoreMesh(core_axis_name="c", subcore_axis_name="s")

    @functools.partial(
        pl.kernel,
        out_type=jax.ShapeDtypeStruct((n, DIM), table.dtype),
        mesh=mesh,
        scratch_types=[
            pltpu.VMEM_SHARED((VOCAB, DIM), jnp.float32),
            pltpu.VMEM((wpw, WIN), jnp.int32),
            pltpu.VMEM((NBUF, WIN, DIM), jnp.float32),
            pltpu.SemaphoreType.DMA,
            pltpu.SemaphoreType.DMA((NBUF,)),
            pltpu.SemaphoreType.DMA((NBUF,)),
        ],
    )
    def gather_kernel(table_hbm, idx_hbm, out_hbm, table_sh, idx_v, bufs,
                      isem, gsem, wsem):
        cid = lax.axis_index("c")
        sid = lax.axis_index("s")
        wid = sid * 2 + cid

        idx_cp = pltpu.async_copy(idx_hbm.at[pl.ds(wid * wpw, wpw)], idx_v, isem)

        @pl.when(sid == 0)
        def _():
            pltpu.sync_copy(table_hbm, table_sh)

        idx_cp.wait()
        plsc.subcore_barrier()

        base = wid * (wpw * WIN)

        @pl.loop(0, wpw, step=NBUF)
        def _(j0):
            gathers = []
            for b in range(NBUF):
                slot = bufs.at[b]

                @pl.when(j0 > 0)
                def _():
                    pltpu.make_async_copy(
                        slot, out_hbm.at[pl.ds(base, WIN)], wsem.at[b]).wait()

                gathers.append(pltpu.async_copy(
                    table_sh.at[idx_v.at[j0 + b]], slot, gsem.at[b]))

            for b in range(NBUF):
                gathers[b].wait()
                pltpu.async_copy(
                    bufs.at[b], out_hbm.at[pl.ds(base + (j0 + b) * WIN, WIN)],
                    wsem.at[b])

        for b in range(NBUF):
            pltpu.make_async_copy(
                bufs.at[b], out_hbm.at[pl.ds(base, WIN)], wsem.at[b]).wait()

    out = gather_kernel(table, idx2d)
    return out.reshape(batch, seq, DIM)
